# baseline (speedup 1.0000x reference)
import functools

import jax
import jax.numpy as jnp
from jax import lax
from jax.experimental import pallas as pl
from jax.experimental.pallas import tpu as pltpu
from jax.experimental.pallas import tpu_sc as plsc

B = 16384
D = 16
NC = 2
NS = 16
NW = NC * NS
BPW = B // NW

RATING_MIN = 1.0
RATING_MAX = 5.0


def _gather4(user, item, t_umlp, t_imlp, t_umf, t_imf):
    fs = jax.ShapeDtypeStruct((B, D), jnp.float32)
    mesh = plsc.VectorSubcoreMesh(core_axis_name="c", subcore_axis_name="s")

    @functools.partial(
        pl.kernel,
        mesh=mesh,
        out_type=(fs, fs, fs, fs),
        compiler_params=pltpu.CompilerParams(use_tc_tiling_on_sc=False),
        scratch_types=[
            pltpu.VMEM((BPW,), jnp.int32),
            pltpu.VMEM((BPW,), jnp.int32),
            pltpu.VMEM((BPW, D), jnp.float32),
            pltpu.VMEM((BPW, D), jnp.float32),
            pltpu.VMEM((BPW, D), jnp.float32),
            pltpu.VMEM((BPW, D), jnp.float32),
            pltpu.SemaphoreType.DMA,
        ],
    )
    def body(user_hbm, item_hbm, umlp_hbm, imlp_hbm, umf_hbm, imf_hbm,
             o_umlp, o_imlp, o_umf, o_imf,
             uidx, iidx, r0, r1, r2, r3, sem):
        wid = lax.axis_index("s") * NC + lax.axis_index("c")
        base = wid * BPW
        pltpu.sync_copy(user_hbm.at[pl.ds(base, BPW)], uidx)
        pltpu.sync_copy(item_hbm.at[pl.ds(base, BPW)], iidx)
        c0 = pltpu.async_copy(umlp_hbm.at[uidx], r0, sem)
        c1 = pltpu.async_copy(imlp_hbm.at[iidx], r1, sem)
        c2 = pltpu.async_copy(umf_hbm.at[uidx], r2, sem)
        c3 = pltpu.async_copy(imf_hbm.at[iidx], r3, sem)
        c0.wait()
        c1.wait()
        c2.wait()
        c3.wait()
        pltpu.sync_copy(r0, o_umlp.at[pl.ds(base, BPW)])
        pltpu.sync_copy(r1, o_imlp.at[pl.ds(base, BPW)])
        pltpu.sync_copy(r2, o_umf.at[pl.ds(base, BPW)])
        pltpu.sync_copy(r3, o_imf.at[pl.ds(base, BPW)])

    return body(user, item, t_umlp, t_imlp, t_umf, t_imf)


BBLK = 2048


def _dense_body(ue_ref, ie_ref, umf_ref, imf_ref, r_ref, w1_ref, w2_ref,
                w3_ref, loss_ref, tgt_ref):
    @pl.when(pl.program_id(0) == 0)
    def _init():
        loss_ref[...] = jnp.zeros((1, 1), dtype=jnp.float32)

    h = jnp.concatenate([ue_ref[...], ie_ref[...]], axis=1)
    h = jnp.tanh(jnp.dot(h, w1_ref[...], preferred_element_type=jnp.float32))
    h = jnp.tanh(jnp.dot(h, w2_ref[...], preferred_element_type=jnp.float32))
    mlp = jnp.tanh(jnp.sum(h * w3_ref[...], axis=1))
    u = umf_ref[...]
    v = imf_ref[...]
    u = u - jnp.mean(u, axis=1, keepdims=True)
    v = v - jnp.mean(v, axis=1, keepdims=True)
    un = jnp.maximum(jnp.sqrt(jnp.sum(u * u, axis=1)), 1e-12)
    vn = jnp.maximum(jnp.sqrt(jnp.sum(v * v, axis=1)), 1e-12)
    mf = jnp.sum(u * v, axis=1) / (un * vn)
    nmf = 0.5 * mlp + 0.5 * mf
    r = (r_ref[...] - RATING_MIN) * (1.0 / (RATING_MAX - RATING_MIN))
    loss_ref[...] += jnp.full((1, 1), jnp.sum((nmf - r) ** 2) * (1.0 / B),
                              dtype=jnp.float32)
    tgt_ref[...] = nmf * (RATING_MAX - RATING_MIN) + RATING_MIN


def _dense(ue, ie, umf, imf, rating, w1t, w2t, w3):
    emb_spec = pl.BlockSpec((BBLK, D), lambda i: (i, 0))
    vec_spec = pl.BlockSpec((BBLK,), lambda i: (i,))
    return pl.pallas_call(
        _dense_body,
        grid=(B // BBLK,),
        in_specs=[
            emb_spec, emb_spec, emb_spec, emb_spec, vec_spec,
            pl.BlockSpec((32, 64), lambda i: (0, 0)),
            pl.BlockSpec((64, 32), lambda i: (0, 0)),
            pl.BlockSpec((1, 32), lambda i: (0, 0)),
        ],
        out_specs=(pl.BlockSpec((1, 1), lambda i: (0, 0)), vec_spec),
        out_shape=(
            jax.ShapeDtypeStruct((1, 1), jnp.float32),
            jax.ShapeDtypeStruct((B,), jnp.float32),
        ),
    )(ue, ie, umf, imf, rating, w1t, w2t, w3)


def kernel(user, item, rating, user_weight_mlp, item_weight_mlp,
           user_weight_mf, item_weight_mf, W1, W2, W3):
    ue, ie, umf, imf = _gather4(user, item, user_weight_mlp, item_weight_mlp,
                                user_weight_mf, item_weight_mf)
    loss2, tgt = _dense(ue, ie, umf, imf, rating, W1.T, W2.T, W3)
    return loss2[0, 0], tgt


# speedup vs baseline: 1.3942x; 1.3942x over previous
import functools

import jax
import jax.numpy as jnp
from jax import lax
from jax.experimental import pallas as pl
from jax.experimental.pallas import tpu as pltpu
from jax.experimental.pallas import tpu_sc as plsc

B = 16384
D = 16
NC = 2
NS = 16
NW = NC * NS
BPW = B // NW
CH = 64
NCHUNK = BPW // CH
NT = 125000

RATING_MIN = 1.0
RATING_MAX = 5.0


HALF = BPW // 2


def _gather_body(user_hbm, item_hbm, umlp_hbm, imlp_hbm, umf_hbm, imf_hbm,
                 o_umlp, o_imlp, o_umf, o_imf,
                 uidx, iidx, buf0, buf1, sem0, sem1):
    wid = lax.axis_index("s") * NC + lax.axis_index("c")
    base = wid * BPW
    pltpu.sync_copy(user_hbm.at[pl.ds(base, BPW)], uidx.at[pl.ds(0, BPW)])
    pltpu.sync_copy(item_hbm.at[pl.ds(base, BPW)], iidx.at[pl.ds(0, BPW)])

    rounds = []
    for table, idx_ref, out in (
        (umlp_hbm, uidx, o_umlp),
        (imlp_hbm, iidx, o_imlp),
        (umf_hbm, uidx, o_umf),
        (imf_hbm, iidx, o_imf),
    ):
        for half in range(2):
            rounds.append((table, idx_ref, out, half))

    def fire(r):
        table, idx_ref, _, half = rounds[r]
        buf, sem = (buf0, sem0) if r % 2 == 0 else (buf1, sem1)

        def one(e, _):
            i = idx_ref[pl.ds(half * HALF + e, 16)][0]
            pltpu.async_copy(table.at[i], buf.at[e], sem)
            return ()

        lax.fori_loop(0, HALF, one, (), unroll=False)

    def drain_and_write(r):
        table, _, out, half = rounds[r]
        buf, sem = (buf0, sem0) if r % 2 == 0 else (buf1, sem1)
        pltpu.make_async_copy(table.at[pl.ds(0, HALF)], buf, sem).wait()
        pltpu.sync_copy(buf, out.at[pl.ds(base + half * HALF, HALF)])

    fire(0)
    for r in range(len(rounds)):
        if r + 1 < len(rounds):
            fire(r + 1)
        drain_and_write(r)


def _gather4(user, item, t_umlp, t_imlp, t_umf, t_imf):
    fs = jax.ShapeDtypeStruct((B, D), jnp.float32)
    mesh = plsc.VectorSubcoreMesh(core_axis_name="c", subcore_axis_name="s")
    f = functools.partial(
        pl.kernel,
        mesh=mesh,
        out_type=(fs, fs, fs, fs),
        compiler_params=pltpu.CompilerParams(
            use_tc_tiling_on_sc=True, needs_layout_passes=False),
        scratch_types=[
            pltpu.VMEM((BPW + 16,), jnp.int32),
            pltpu.VMEM((BPW + 16,), jnp.int32),
            pltpu.VMEM((HALF, D), jnp.float32),
            pltpu.VMEM((HALF, D), jnp.float32),
            pltpu.SemaphoreType.DMA,
            pltpu.SemaphoreType.DMA,
        ],
    )(_gather_body)
    return f(user, item, t_umlp, t_imlp, t_umf, t_imf)


BBLK = 2048


def _dense_body(ue_ref, ie_ref, umf_ref, imf_ref, r_ref, w1_ref, w2_ref,
                w3_ref, loss_ref, tgt_ref):
    @pl.when(pl.program_id(0) == 0)
    def _init():
        loss_ref[...] = jnp.zeros((1, 1), dtype=jnp.float32)

    h = jnp.concatenate([ue_ref[...], ie_ref[...]], axis=1)
    h = jnp.tanh(jnp.dot(h, w1_ref[...], preferred_element_type=jnp.float32))
    h = jnp.tanh(jnp.dot(h, w2_ref[...], preferred_element_type=jnp.float32))
    mlp = jnp.tanh(jnp.sum(h * w3_ref[...], axis=1))
    u = umf_ref[...]
    v = imf_ref[...]
    u = u - jnp.mean(u, axis=1, keepdims=True)
    v = v - jnp.mean(v, axis=1, keepdims=True)
    un = jnp.maximum(jnp.sqrt(jnp.sum(u * u, axis=1)), 1e-12)
    vn = jnp.maximum(jnp.sqrt(jnp.sum(v * v, axis=1)), 1e-12)
    mf = jnp.sum(u * v, axis=1) / (un * vn)
    nmf = 0.5 * mlp + 0.5 * mf
    r = (r_ref[...] - RATING_MIN) * (1.0 / (RATING_MAX - RATING_MIN))
    loss_ref[...] += jnp.full((1, 1), jnp.sum((nmf - r) ** 2) * (1.0 / B),
                              dtype=jnp.float32)
    tgt_ref[...] = nmf * (RATING_MAX - RATING_MIN) + RATING_MIN


def _dense(ue, ie, umf, imf, rating, w1t, w2t, w3):
    emb_spec = pl.BlockSpec((BBLK, D), lambda i: (i, 0))
    vec_spec = pl.BlockSpec((BBLK,), lambda i: (i,))
    return pl.pallas_call(
        _dense_body,
        grid=(B // BBLK,),
        in_specs=[
            emb_spec, emb_spec, emb_spec, emb_spec, vec_spec,
            pl.BlockSpec((32, 64), lambda i: (0, 0)),
            pl.BlockSpec((64, 32), lambda i: (0, 0)),
            pl.BlockSpec((1, 32), lambda i: (0, 0)),
        ],
        out_specs=(pl.BlockSpec((1, 1), lambda i: (0, 0)), vec_spec),
        out_shape=(
            jax.ShapeDtypeStruct((1, 1), jnp.float32),
            jax.ShapeDtypeStruct((B,), jnp.float32),
        ),
    )(ue, ie, umf, imf, rating, w1t, w2t, w3)


def kernel(user, item, rating, user_weight_mlp, item_weight_mlp,
           user_weight_mf, item_weight_mf, W1, W2, W3):
    ue, ie, umf, imf = _gather4(user, item, user_weight_mlp, item_weight_mlp,
                                user_weight_mf, item_weight_mf)
    loss2, tgt = _dense(ue, ie, umf, imf, rating, W1.T, W2.T, W3)
    return loss2[0, 0], tgt


# speedup vs baseline: 5.6196x; 4.0307x over previous
import functools

import jax
import jax.numpy as jnp
from jax import lax
from jax.experimental import pallas as pl
from jax.experimental.pallas import tpu as pltpu
from jax.experimental.pallas import tpu_sc as plsc

B = 16384
D = 16
NC = 2
NS = 16
NW = NC * NS
BPW = B // NW

RATING_MIN = 1.0
RATING_MAX = 5.0


RB = 8


def _gather_body(user_hbm, item_hbm, umlp_hbm, imlp_hbm, umf_hbm, imf_hbm,
                 out_hbm, uidx, iidx, stage, rbufs, sems):
    wid = lax.axis_index("s") * NC + lax.axis_index("c")
    base = wid * BPW
    pltpu.sync_copy(user_hbm.at[pl.ds(base, BPW)], uidx.at[pl.ds(0, BPW)])
    pltpu.sync_copy(item_hbm.at[pl.ds(base, BPW)], iidx.at[pl.ds(0, BPW)])
    iota = lax.iota(jnp.int32, 16)

    for t, (table, idx_ref) in enumerate((
        (umlp_hbm, uidx),
        (imlp_hbm, iidx),
        (umf_hbm, uidx),
        (imf_hbm, iidx),
    )):
        def fire(e, k, table=table, idx_ref=idx_ref):
            i = idx_ref[pl.ds(e, 16)][0]
            col = pl.multiple_of(lax.shift_right_logical(i, 7) * 128, 128)
            pltpu.async_copy(table.at[:, pl.ds(col, 128)], rbufs[k], sems[k])

        def wait(k, table=table):
            pltpu.make_async_copy(
                table.at[:, pl.ds(0, 128)], rbufs[k], sems[k]).wait()

        def extract(e, k, idx_ref=idx_ref, t=t):
            i = idx_ref[pl.ds(e, 16)][0]
            lane = jnp.broadcast_to(jnp.bitwise_and(i, 127), (16,))
            vals = plsc.load_gather(rbufs[k], [iota, lane])
            plsc.store_scatter(
                stage, [iota + t * D, jnp.broadcast_to(e, (16,))], vals)

        for k in range(RB):
            fire(k, k)

        def group(g, _):
            for k in range(RB):
                e = g * RB + k
                wait(k)
                extract(e, k)
                fire(e + RB, k)
            return ()

        lax.fori_loop(0, BPW // RB - 1, group, (), unroll=False)
        for k in range(RB):
            e = BPW - RB + k
            wait(k)
            extract(e, k)

    pltpu.sync_copy(stage, out_hbm.at[:, pl.ds(base, BPW)])


def _gather4(user, item, t_umlp, t_imlp, t_umf, t_imf):
    mesh = plsc.VectorSubcoreMesh(core_axis_name="c", subcore_axis_name="s")
    f = functools.partial(
        pl.kernel,
        mesh=mesh,
        out_type=jax.ShapeDtypeStruct((4 * D, B), jnp.float32),
        compiler_params=pltpu.CompilerParams(
            use_tc_tiling_on_sc=True, needs_layout_passes=False),
        scratch_types=[
            pltpu.VMEM((BPW + 16,), jnp.int32),
            pltpu.VMEM((BPW + 16,), jnp.int32),
            pltpu.VMEM((4 * D, BPW), jnp.float32),
            [pltpu.VMEM((D, 128), jnp.float32) for _ in range(RB)],
            [pltpu.SemaphoreType.DMA for _ in range(RB)],
        ],
    )(_gather_body)
    return f(user, item, t_umlp.T, t_imlp.T, t_umf.T, t_imf.T)


BBLK = 16384


def _dense_body(x_ref, r_ref, w1_ref, w2_ref, w3_ref, loss_ref, tgt_ref):
    @pl.when(pl.program_id(0) == 0)
    def _init():
        loss_ref[...] = jnp.zeros((1, 1), dtype=jnp.float32)

    h = x_ref[pl.ds(0, 2 * D), :]
    h = jnp.tanh(jnp.dot(w1_ref[...], h, preferred_element_type=jnp.float32))
    h = jnp.tanh(jnp.dot(w2_ref[...], h, preferred_element_type=jnp.float32))
    mlp = jnp.tanh(jnp.dot(w3_ref[...], h, preferred_element_type=jnp.float32))
    u = x_ref[pl.ds(2 * D, D), :]
    v = x_ref[pl.ds(3 * D, D), :]
    u = u - jnp.mean(u, axis=0, keepdims=True)
    v = v - jnp.mean(v, axis=0, keepdims=True)
    un = jnp.maximum(jnp.sqrt(jnp.sum(u * u, axis=0, keepdims=True)), 1e-12)
    vn = jnp.maximum(jnp.sqrt(jnp.sum(v * v, axis=0, keepdims=True)), 1e-12)
    mf = jnp.sum(u * v, axis=0, keepdims=True) / (un * vn)
    nmf = 0.5 * mlp + 0.5 * mf
    r = (r_ref[...] - RATING_MIN) * (1.0 / (RATING_MAX - RATING_MIN))
    loss_ref[...] += jnp.full((1, 1), jnp.sum((nmf - r) ** 2) * (1.0 / B),
                              dtype=jnp.float32)
    tgt_ref[...] = nmf * (RATING_MAX - RATING_MIN) + RATING_MIN


def _dense(x, rating2, w1, w2, w3):
    return pl.pallas_call(
        _dense_body,
        grid=(B // BBLK,),
        in_specs=[
            pl.BlockSpec((4 * D, BBLK), lambda i: (0, i)),
            pl.BlockSpec((1, BBLK), lambda i: (0, i)),
            pl.BlockSpec((64, 32), lambda i: (0, 0)),
            pl.BlockSpec((32, 64), lambda i: (0, 0)),
            pl.BlockSpec((1, 32), lambda i: (0, 0)),
        ],
        out_specs=(
            pl.BlockSpec((1, 1), lambda i: (0, 0)),
            pl.BlockSpec((1, BBLK), lambda i: (0, i)),
        ),
        out_shape=(
            jax.ShapeDtypeStruct((1, 1), jnp.float32),
            jax.ShapeDtypeStruct((1, B), jnp.float32),
        ),
    )(x, rating2, w1, w2, w3)


def kernel(user, item, rating, user_weight_mlp, item_weight_mlp,
           user_weight_mf, item_weight_mf, W1, W2, W3):
    x = _gather4(user, item, user_weight_mlp, item_weight_mlp,
                 user_weight_mf, item_weight_mf)
    loss2, tgt2 = _dense(x, rating.reshape(1, B), W1, W2, W3)
    return loss2[0, 0], tgt2.reshape(B)
